# Initial kernel scaffold; baseline (speedup 1.0000x reference)
import jax
import jax.numpy as jnp
from jax.experimental import pallas as pl


def kernel(x, ln1_g, ln1_b, in_proj_w, in_proj_b, out_proj_w, out_proj_b, ln2_g, ln2_b, gate_w, gate_b, w_up, b_up, w_down, b_down):
    raise NotImplementedError("write your pallas kernel here")


# speedup vs baseline: 1.5890x; 1.5890x over previous
import jax
import jax.numpy as jnp
import numpy as np
from jax.experimental import pallas as pl
from jax.experimental.pallas import tpu as pltpu

S, D, H, NT = 2048, 768, 12, 8
DFF = 4 * D
DH = D // H
TM = 256
TQ = 512
NB = S // TM + NT - 1
EPS = 1e-5
F32 = jnp.float32
BF16 = jnp.bfloat16


def _ln(x, g, b):
    m = jnp.mean(x, axis=-1, keepdims=True)
    v = jnp.mean((x - m) ** 2, axis=-1, keepdims=True)
    return (x - m) / jnp.sqrt(v + EPS) * g + b


def _qkv_kernel(x_ref, g_ref, b_ref, w_ref, wb_ref, qkv_ref):
    n = _ln(x_ref[...], g_ref[...], b_ref[...])
    qkv_ref[...] = jnp.dot(n, w_ref[...], preferred_element_type=F32) + wb_ref[...]


def _attn_kernel(q_ref, k_ref, v_ref, o_ref):
    s = jax.lax.dot_general(q_ref[0], k_ref[0], (((1,), (1,)), ((), ())),
                            preferred_element_type=F32) * (1.0 / np.sqrt(DH))
    m = jnp.max(s, axis=-1, keepdims=True)
    p = jnp.exp(s - m)
    p = p / jnp.sum(p, axis=-1, keepdims=True)
    o_ref[0] = jnp.dot(p, v_ref[0], preferred_element_type=F32)


def _post_kernel(o_ref, x_ref, wo_ref, bo_ref, g2_ref, b2_ref, wg_ref, bg_ref,
                 x1_ref, n2_ref, gate_ref):
    a = jnp.dot(o_ref[...], wo_ref[...], preferred_element_type=F32) + bo_ref[...]
    x1 = x_ref[...] + a
    x1_ref[...] = x1
    n2 = _ln(x1, g2_ref[...], b2_ref[...])
    n2_ref[...] = n2
    logits = jnp.dot(n2, wg_ref[...], preferred_element_type=F32) + bg_ref[...]
    mx = jnp.max(logits, axis=-1, keepdims=True)
    iot = jax.lax.broadcasted_iota(jnp.int32, logits.shape, 1)
    first = jnp.min(jnp.where(logits >= mx, iot, NT), axis=-1, keepdims=True)
    gate_ref[...] = (iot == first).astype(F32)


def _ffn_kernel(be_ref, gidx_ref, sidx_ref, nblk_ref,
                n2_ref, x1_ref, wu_ref, bu_ref, wd_ref, bd_ref,
                out_ref, xg_ref, y_ref):
    del be_ref
    b = pl.program_id(0)

    @pl.when(b < nblk_ref[0])
    def _():
        base = b * TM

        def gather(i, _):
            t = gidx_ref[base + i]
            xg_ref[pl.ds(i, 1), :] = n2_ref[pl.ds(t, 1), :]
            return 0

        jax.lax.fori_loop(0, TM, gather, 0, unroll=8)

        h = jnp.maximum(
            jnp.dot(xg_ref[...].astype(BF16), wu_ref[0],
                    preferred_element_type=F32) + bu_ref[0], 0.0)
        y_ref[...] = (jnp.dot(h.astype(BF16), wd_ref[0],
                              preferred_element_type=F32) + bd_ref[0])

        def scatter(i, _):
            t = sidx_ref[base + i]

            @pl.when(t < S)
            def _():
                out_ref[pl.ds(t, 1), :] = (
                    y_ref[pl.ds(i, 1), :] + x1_ref[pl.ds(t, 1), :])
            return 0

        jax.lax.fori_loop(0, TM, scatter, 0, unroll=8)


def kernel(x, ln1_g, ln1_b, in_proj_w, in_proj_b, out_proj_w, out_proj_b,
           ln2_g, ln2_b, gate_w, gate_b, w_up, b_up, w_down, b_down):
    xf = x.reshape(S, D)
    wqkv = in_proj_w.T
    wo = out_proj_w.T
    wg = gate_w.T
    wu = w_up.astype(BF16)
    wd = w_down.astype(BF16)
    bu = b_up.reshape(NT, 1, DFF)
    bd = b_down.reshape(NT, 1, D)
    g1 = ln1_g.reshape(1, D); b1 = ln1_b.reshape(1, D)
    g2 = ln2_g.reshape(1, D); b2 = ln2_b.reshape(1, D)
    bqkv = in_proj_b.reshape(1, 3 * D)
    bo = out_proj_b.reshape(1, D)
    bg = gate_b.reshape(1, NT)

    qkv = pl.pallas_call(
        _qkv_kernel,
        grid=(S // TM,),
        in_specs=[
            pl.BlockSpec((TM, D), lambda i: (i, 0)),
            pl.BlockSpec((1, D), lambda i: (0, 0)),
            pl.BlockSpec((1, D), lambda i: (0, 0)),
            pl.BlockSpec((D, 3 * D), lambda i: (0, 0)),
            pl.BlockSpec((1, 3 * D), lambda i: (0, 0)),
        ],
        out_specs=pl.BlockSpec((TM, 3 * D), lambda i: (i, 0)),
        out_shape=jax.ShapeDtypeStruct((S, 3 * D), F32),
    )(xf, g1, b1, wqkv, bqkv)

    q = qkv[:, :D].reshape(S, H, DH).transpose(1, 0, 2)
    k = qkv[:, D:2 * D].reshape(S, H, DH).transpose(1, 0, 2)
    v = qkv[:, 2 * D:].reshape(S, H, DH).transpose(1, 0, 2)
    o = pl.pallas_call(
        _attn_kernel,
        grid=(H, S // TQ),
        in_specs=[
            pl.BlockSpec((1, TQ, DH), lambda h, i: (h, i, 0)),
            pl.BlockSpec((1, S, DH), lambda h, i: (h, 0, 0)),
            pl.BlockSpec((1, S, DH), lambda h, i: (h, 0, 0)),
        ],
        out_specs=pl.BlockSpec((1, TQ, DH), lambda h, i: (h, i, 0)),
        out_shape=jax.ShapeDtypeStruct((H, S, DH), F32),
    )(q, k, v)
    of = o.transpose(1, 0, 2).reshape(S, D)

    x1, n2, gate = pl.pallas_call(
        _post_kernel,
        grid=(S // TM,),
        in_specs=[
            pl.BlockSpec((TM, D), lambda i: (i, 0)),
            pl.BlockSpec((TM, D), lambda i: (i, 0)),
            pl.BlockSpec((D, D), lambda i: (0, 0)),
            pl.BlockSpec((1, D), lambda i: (0, 0)),
            pl.BlockSpec((1, D), lambda i: (0, 0)),
            pl.BlockSpec((1, D), lambda i: (0, 0)),
            pl.BlockSpec((D, NT), lambda i: (0, 0)),
            pl.BlockSpec((1, NT), lambda i: (0, 0)),
        ],
        out_specs=[
            pl.BlockSpec((TM, D), lambda i: (i, 0)),
            pl.BlockSpec((TM, D), lambda i: (i, 0)),
            pl.BlockSpec((TM, NT), lambda i: (i, 0)),
        ],
        out_shape=[
            jax.ShapeDtypeStruct((S, D), F32),
            jax.ShapeDtypeStruct((S, D), F32),
            jax.ShapeDtypeStruct((S, NT), F32),
        ],
    )(of, xf, wo, bo, g2, b2, wg, bg)

    e = jnp.argmax(gate, axis=-1).astype(jnp.int32)
    counts = jnp.sum(gate, axis=0).astype(jnp.int32)
    blocks_t = (counts + TM - 1) // TM
    ends = jnp.cumsum(blocks_t)
    blk_start = ends - blocks_t
    nblk = ends[-1].astype(jnp.int32)
    ranks = jnp.cumsum(gate, axis=0) - gate
    r = jnp.sum(ranks * gate, axis=1).astype(jnp.int32)
    slot = blk_start[e] * TM + r
    tok = jnp.arange(S, dtype=jnp.int32)
    gather_tok = jnp.zeros((NB * TM,), jnp.int32).at[slot].set(tok)
    scatter_tok = jnp.full((NB * TM,), S, jnp.int32).at[slot].set(tok)
    be = jnp.searchsorted(ends, jnp.arange(NB, dtype=jnp.int32), side='right')
    be = jnp.minimum(be, NT - 1).astype(jnp.int32)
    be_last = jnp.take(be, nblk - 1)
    be = jnp.where(jnp.arange(NB) < nblk, be, be_last)
    nblk_arr = nblk.reshape(1)

    grid_spec = pltpu.PrefetchScalarGridSpec(
        num_scalar_prefetch=4,
        grid=(NB,),
        in_specs=[
            pl.BlockSpec((S, D), lambda b, be, gi, si, nb: (0, 0)),
            pl.BlockSpec((S, D), lambda b, be, gi, si, nb: (0, 0)),
            pl.BlockSpec((1, D, DFF), lambda b, be, gi, si, nb: (be[b], 0, 0)),
            pl.BlockSpec((1, 1, DFF), lambda b, be, gi, si, nb: (be[b], 0, 0)),
            pl.BlockSpec((1, DFF, D), lambda b, be, gi, si, nb: (be[b], 0, 0)),
            pl.BlockSpec((1, 1, D), lambda b, be, gi, si, nb: (be[b], 0, 0)),
        ],
        out_specs=pl.BlockSpec((S, D), lambda b, be, gi, si, nb: (0, 0)),
        scratch_shapes=[
            pltpu.VMEM((TM, D), F32),
            pltpu.VMEM((TM, D), F32),
        ],
    )
    out = pl.pallas_call(
        _ffn_kernel,
        grid_spec=grid_spec,
        out_shape=jax.ShapeDtypeStruct((S, D), F32),
        compiler_params=pltpu.CompilerParams(
            dimension_semantics=("arbitrary",)),
    )(be, gather_tok, scatter_tok, nblk_arr,
      n2, x1, wu, bu, wd, bd)

    return out.reshape(1, S, D), gate.reshape(1, S, NT)


# speedup vs baseline: 1.8682x; 1.1757x over previous
import jax
import jax.numpy as jnp
import numpy as np
from jax.experimental import pallas as pl
from jax.experimental.pallas import tpu as pltpu

S, D, H, NT = 2048, 768, 12, 8
DFF = 4 * D
DH = D // H
TM = 256
TQ = 512
NB = S // TM + NT - 1
EPS = 1e-5
F32 = jnp.float32
BF16 = jnp.bfloat16


def _ln(x, g, b):
    m = jnp.mean(x, axis=-1, keepdims=True)
    v = jnp.mean((x - m) ** 2, axis=-1, keepdims=True)
    return (x - m) / jnp.sqrt(v + EPS) * g + b


def _qkv_kernel(x_ref, g_ref, b_ref, w_ref, wb_ref, qkv_ref):
    n = _ln(x_ref[...], g_ref[...], b_ref[...])
    qkv_ref[...] = jax.lax.dot_general(
        n, w_ref[...], (((1,), (1,)), ((), ())),
        preferred_element_type=F32) + wb_ref[...]


def _attn_kernel(q_ref, k_ref, v_ref, o_ref):
    s = jax.lax.dot_general(q_ref[0], k_ref[0], (((1,), (1,)), ((), ())),
                            preferred_element_type=F32) * (1.0 / np.sqrt(DH))
    m = jnp.max(s, axis=-1, keepdims=True)
    p = jnp.exp(s - m)
    p = p / jnp.sum(p, axis=-1, keepdims=True)
    o_ref[0] = jnp.dot(p, v_ref[0], preferred_element_type=F32)


def _post_kernel(o_ref, x_ref, wo_ref, bo_ref, g2_ref, b2_ref, wg_ref, bg_ref,
                 x1_ref, n2_ref, gate_ref):
    a = jax.lax.dot_general(o_ref[...], wo_ref[...], (((1,), (1,)), ((), ())),
                            preferred_element_type=F32) + bo_ref[...]
    x1 = x_ref[...] + a
    x1_ref[...] = x1
    n2 = _ln(x1, g2_ref[...], b2_ref[...])
    n2_ref[...] = n2
    logits = jax.lax.dot_general(n2, wg_ref[...], (((1,), (1,)), ((), ())),
                                 preferred_element_type=F32) + bg_ref[...]
    mx = jnp.max(logits, axis=-1, keepdims=True)
    iot = jax.lax.broadcasted_iota(jnp.int32, logits.shape, 1)
    first = jnp.min(jnp.where(logits >= mx, iot, NT), axis=-1, keepdims=True)
    gate_ref[...] = (iot == first).astype(F32)


def _ffn_kernel(be_ref, gidx_ref, sidx_ref, nblk_ref,
                n2_ref, x1_ref, wu_ref, bu_ref, wd_ref, bd_ref,
                out_ref, xg_ref, y_ref):
    del be_ref
    b = pl.program_id(0)

    @pl.when(b < nblk_ref[0])
    def _():
        base = b * TM

        def gather(i, _):
            t = gidx_ref[base + i]
            xg_ref[pl.ds(i, 1), :] = n2_ref[pl.ds(t, 1), :]
            return 0

        jax.lax.fori_loop(0, TM, gather, 0, unroll=8)

        h = jnp.maximum(
            jnp.dot(xg_ref[...], wu_ref[0],
                    preferred_element_type=F32) + bu_ref[0], 0.0)
        y_ref[...] = (jnp.dot(h, wd_ref[0],
                              preferred_element_type=F32) + bd_ref[0])

        def scatter(i, _):
            t = sidx_ref[base + i]

            @pl.when(t < S)
            def _():
                out_ref[pl.ds(t, 1), :] = (
                    y_ref[pl.ds(i, 1), :] + x1_ref[pl.ds(t, 1), :])
            return 0

        jax.lax.fori_loop(0, TM, scatter, 0, unroll=8)


def kernel(x, ln1_g, ln1_b, in_proj_w, in_proj_b, out_proj_w, out_proj_b,
           ln2_g, ln2_b, gate_w, gate_b, w_up, b_up, w_down, b_down):
    xf = x.reshape(S, D)
    wqkv = in_proj_w
    wo = out_proj_w
    wg = gate_w
    wu = w_up
    wd = w_down
    bu = b_up.reshape(NT, 1, DFF)
    bd = b_down.reshape(NT, 1, D)
    g1 = ln1_g.reshape(1, D); b1 = ln1_b.reshape(1, D)
    g2 = ln2_g.reshape(1, D); b2 = ln2_b.reshape(1, D)
    bqkv = in_proj_b.reshape(1, 3 * D)
    bo = out_proj_b.reshape(1, D)
    bg = gate_b.reshape(1, NT)

    qkv = pl.pallas_call(
        _qkv_kernel,
        grid=(S // TM,),
        in_specs=[
            pl.BlockSpec((TM, D), lambda i: (i, 0)),
            pl.BlockSpec((1, D), lambda i: (0, 0)),
            pl.BlockSpec((1, D), lambda i: (0, 0)),
            pl.BlockSpec((3 * D, D), lambda i: (0, 0)),
            pl.BlockSpec((1, 3 * D), lambda i: (0, 0)),
        ],
        out_specs=pl.BlockSpec((TM, 3 * D), lambda i: (i, 0)),
        out_shape=jax.ShapeDtypeStruct((S, 3 * D), F32),
    )(xf, g1, b1, wqkv, bqkv)

    q = qkv[:, :D].reshape(S, H, DH).transpose(1, 0, 2)
    k = qkv[:, D:2 * D].reshape(S, H, DH).transpose(1, 0, 2)
    v = qkv[:, 2 * D:].reshape(S, H, DH).transpose(1, 0, 2)
    o = pl.pallas_call(
        _attn_kernel,
        grid=(H, S // TQ),
        in_specs=[
            pl.BlockSpec((1, TQ, DH), lambda h, i: (h, i, 0)),
            pl.BlockSpec((1, S, DH), lambda h, i: (h, 0, 0)),
            pl.BlockSpec((1, S, DH), lambda h, i: (h, 0, 0)),
        ],
        out_specs=pl.BlockSpec((1, TQ, DH), lambda h, i: (h, i, 0)),
        out_shape=jax.ShapeDtypeStruct((H, S, DH), F32),
    )(q, k, v)
    of = o.transpose(1, 0, 2).reshape(S, D)

    x1, n2, gate = pl.pallas_call(
        _post_kernel,
        grid=(S // TM,),
        in_specs=[
            pl.BlockSpec((TM, D), lambda i: (i, 0)),
            pl.BlockSpec((TM, D), lambda i: (i, 0)),
            pl.BlockSpec((D, D), lambda i: (0, 0)),
            pl.BlockSpec((1, D), lambda i: (0, 0)),
            pl.BlockSpec((1, D), lambda i: (0, 0)),
            pl.BlockSpec((1, D), lambda i: (0, 0)),
            pl.BlockSpec((NT, D), lambda i: (0, 0)),
            pl.BlockSpec((1, NT), lambda i: (0, 0)),
        ],
        out_specs=[
            pl.BlockSpec((TM, D), lambda i: (i, 0)),
            pl.BlockSpec((TM, D), lambda i: (i, 0)),
            pl.BlockSpec((TM, NT), lambda i: (i, 0)),
        ],
        out_shape=[
            jax.ShapeDtypeStruct((S, D), F32),
            jax.ShapeDtypeStruct((S, D), F32),
            jax.ShapeDtypeStruct((S, NT), F32),
        ],
    )(of, xf, wo, bo, g2, b2, wg, bg)

    e = jnp.argmax(gate, axis=-1).astype(jnp.int32)
    counts = jnp.sum(gate, axis=0).astype(jnp.int32)
    blocks_t = (counts + TM - 1) // TM
    ends = jnp.cumsum(blocks_t)
    blk_start = ends - blocks_t
    nblk = ends[-1].astype(jnp.int32)
    ranks = jnp.cumsum(gate, axis=0) - gate
    r = jnp.sum(ranks * gate, axis=1).astype(jnp.int32)
    slot = blk_start[e] * TM + r
    tok = jnp.arange(S, dtype=jnp.int32)
    gather_tok = jnp.zeros((NB * TM,), jnp.int32).at[slot].set(tok)
    scatter_tok = jnp.full((NB * TM,), S, jnp.int32).at[slot].set(tok)
    be = jnp.searchsorted(ends, jnp.arange(NB, dtype=jnp.int32), side='right')
    be = jnp.minimum(be, NT - 1).astype(jnp.int32)
    be_last = jnp.take(be, nblk - 1)
    be = jnp.where(jnp.arange(NB) < nblk, be, be_last)
    nblk_arr = nblk.reshape(1)

    grid_spec = pltpu.PrefetchScalarGridSpec(
        num_scalar_prefetch=4,
        grid=(NB,),
        in_specs=[
            pl.BlockSpec((S, D), lambda b, be, gi, si, nb: (0, 0)),
            pl.BlockSpec((S, D), lambda b, be, gi, si, nb: (0, 0)),
            pl.BlockSpec((1, D, DFF), lambda b, be, gi, si, nb: (be[b], 0, 0)),
            pl.BlockSpec((1, 1, DFF), lambda b, be, gi, si, nb: (be[b], 0, 0)),
            pl.BlockSpec((1, DFF, D), lambda b, be, gi, si, nb: (be[b], 0, 0)),
            pl.BlockSpec((1, 1, D), lambda b, be, gi, si, nb: (be[b], 0, 0)),
        ],
        out_specs=pl.BlockSpec((S, D), lambda b, be, gi, si, nb: (0, 0)),
        scratch_shapes=[
            pltpu.VMEM((TM, D), F32),
            pltpu.VMEM((TM, D), F32),
        ],
    )
    out = pl.pallas_call(
        _ffn_kernel,
        grid_spec=grid_spec,
        out_shape=jax.ShapeDtypeStruct((S, D), F32),
        compiler_params=pltpu.CompilerParams(
            dimension_semantics=("arbitrary",)),
    )(be, gather_tok, scatter_tok, nblk_arr,
      n2, x1, wu, bu, wd, bd)

    return out.reshape(1, S, D), gate.reshape(1, S, NT)


# speedup vs baseline: 2.0471x; 1.0958x over previous
import functools

import jax
import jax.numpy as jnp
import numpy as np
from jax import lax
from jax.experimental import pallas as pl
from jax.experimental.pallas import tpu as pltpu
from jax.experimental.pallas import tpu_sc as plsc

S, D, H, NT = 2048, 768, 12, 8
DFF = 4 * D
DH = D // H
TM = 256
TQ = 512
NB = S // TM + NT - 1
EPS = 1e-5
F32 = jnp.float32

_SC_INFO = plsc.get_sparse_core_info()
_NW = _SC_INFO.num_cores * _SC_INFO.num_subcores
_TPW = S // _NW


def _ln(x, g, b):
    m = jnp.mean(x, axis=-1, keepdims=True)
    v = jnp.mean((x - m) ** 2, axis=-1, keepdims=True)
    return (x - m) / jnp.sqrt(v + EPS) * g + b


def _qkv_kernel(x_ref, g_ref, b_ref, w_ref, wb_ref, qkv_ref):
    n = _ln(x_ref[...], g_ref[...], b_ref[...])
    qkv_ref[...] = jax.lax.dot_general(
        n, w_ref[...], (((1,), (1,)), ((), ())),
        preferred_element_type=F32) + wb_ref[...]


def _attn_kernel(q_ref, k_ref, v_ref, o_ref):
    s = jax.lax.dot_general(q_ref[0], k_ref[0], (((1,), (1,)), ((), ())),
                            preferred_element_type=F32) * (1.0 / np.sqrt(DH))
    m = jnp.max(s, axis=-1, keepdims=True)
    p = jnp.exp(s - m)
    p = p / jnp.sum(p, axis=-1, keepdims=True)
    o_ref[0] = jnp.dot(p, v_ref[0], preferred_element_type=F32)


def _post_kernel(o_ref, x_ref, wo_ref, bo_ref, g2_ref, b2_ref, wg_ref, bg_ref,
                 x1_ref, n2_ref, gate_ref):
    a = jax.lax.dot_general(o_ref[...], wo_ref[...], (((1,), (1,)), ((), ())),
                            preferred_element_type=F32) + bo_ref[...]
    x1 = x_ref[...] + a
    x1_ref[...] = x1
    n2 = _ln(x1, g2_ref[...], b2_ref[...])
    n2_ref[...] = n2
    logits = jax.lax.dot_general(n2, wg_ref[...], (((1,), (1,)), ((), ())),
                                 preferred_element_type=F32) + bg_ref[...]
    mx = jnp.max(logits, axis=-1, keepdims=True)
    iot = jax.lax.broadcasted_iota(jnp.int32, logits.shape, 1)
    first = jnp.min(jnp.where(logits >= mx, iot, NT), axis=-1, keepdims=True)
    gate_ref[...] = (iot == first).astype(F32)


def _dispatch_sc(n2, x1, slot):
    mesh = plsc.VectorSubcoreMesh(core_axis_name="c", subcore_axis_name="s")

    @functools.partial(
        pl.kernel, mesh=mesh,
        out_type=[jax.ShapeDtypeStruct((NB * TM, D), F32),
                  jax.ShapeDtypeStruct((NB * TM, D), F32)],
        scratch_types=[pltpu.VMEM((_TPW,), jnp.int32),
                       pltpu.VMEM((_TPW, D), F32),
                       pltpu.SemaphoreType.DMA],
    )
    def k(n2_hbm, x1_hbm, slot_hbm, xg_hbm, x1g_hbm, idx_v, rows_v, sem):
        wid = lax.axis_index("s") * _SC_INFO.num_cores + lax.axis_index("c")
        base = wid * _TPW
        pltpu.sync_copy(slot_hbm.at[pl.ds(base, _TPW)], idx_v)
        pltpu.sync_copy(n2_hbm.at[pl.ds(base, _TPW)], rows_v)
        pltpu.async_copy(rows_v, xg_hbm.at[idx_v], sem).wait()
        pltpu.sync_copy(x1_hbm.at[pl.ds(base, _TPW)], rows_v)
        pltpu.async_copy(rows_v, x1g_hbm.at[idx_v], sem).wait()

    return k(n2, x1, slot)


def _combine_sc(yg, slot):
    mesh = plsc.VectorSubcoreMesh(core_axis_name="c", subcore_axis_name="s")

    @functools.partial(
        pl.kernel, mesh=mesh,
        out_type=jax.ShapeDtypeStruct((S, D), F32),
        scratch_types=[pltpu.VMEM((_TPW,), jnp.int32),
                       pltpu.VMEM((_TPW, D), F32),
                       pltpu.SemaphoreType.DMA],
    )
    def k(yg_hbm, slot_hbm, out_hbm, idx_v, rows_v, sem):
        wid = lax.axis_index("s") * _SC_INFO.num_cores + lax.axis_index("c")
        base = wid * _TPW
        pltpu.sync_copy(slot_hbm.at[pl.ds(base, _TPW)], idx_v)
        pltpu.async_copy(yg_hbm.at[idx_v], rows_v, sem).wait()
        pltpu.sync_copy(rows_v, out_hbm.at[pl.ds(base, _TPW)])

    return k(yg, slot)


def _ffn_kernel(be_ref, nblk_ref, xg_ref, x1g_ref,
                wu_ref, bu_ref, wd_ref, bd_ref, yg_ref):
    del be_ref
    b = pl.program_id(0)

    @pl.when(b < nblk_ref[0])
    def _():
        h = jnp.maximum(
            jnp.dot(xg_ref[...], wu_ref[0], preferred_element_type=F32)
            + bu_ref[0], 0.0)
        yg_ref[...] = (jnp.dot(h, wd_ref[0], preferred_element_type=F32)
                       + bd_ref[0] + x1g_ref[...])


def kernel(x, ln1_g, ln1_b, in_proj_w, in_proj_b, out_proj_w, out_proj_b,
           ln2_g, ln2_b, gate_w, gate_b, w_up, b_up, w_down, b_down):
    xf = x.reshape(S, D)
    bu = b_up.reshape(NT, 1, DFF)
    bd = b_down.reshape(NT, 1, D)
    g1 = ln1_g.reshape(1, D); b1 = ln1_b.reshape(1, D)
    g2 = ln2_g.reshape(1, D); b2 = ln2_b.reshape(1, D)
    bqkv = in_proj_b.reshape(1, 3 * D)
    bo = out_proj_b.reshape(1, D)
    bg = gate_b.reshape(1, NT)

    qkv = pl.pallas_call(
        _qkv_kernel,
        grid=(S // TM,),
        in_specs=[
            pl.BlockSpec((TM, D), lambda i: (i, 0)),
            pl.BlockSpec((1, D), lambda i: (0, 0)),
            pl.BlockSpec((1, D), lambda i: (0, 0)),
            pl.BlockSpec((3 * D, D), lambda i: (0, 0)),
            pl.BlockSpec((1, 3 * D), lambda i: (0, 0)),
        ],
        out_specs=pl.BlockSpec((TM, 3 * D), lambda i: (i, 0)),
        out_shape=jax.ShapeDtypeStruct((S, 3 * D), F32),
    )(xf, g1, b1, in_proj_w, bqkv)

    q = qkv[:, :D].reshape(S, H, DH).transpose(1, 0, 2)
    k = qkv[:, D:2 * D].reshape(S, H, DH).transpose(1, 0, 2)
    v = qkv[:, 2 * D:].reshape(S, H, DH).transpose(1, 0, 2)
    o = pl.pallas_call(
        _attn_kernel,
        grid=(H, S // TQ),
        in_specs=[
            pl.BlockSpec((1, TQ, DH), lambda h, i: (h, i, 0)),
            pl.BlockSpec((1, S, DH), lambda h, i: (h, 0, 0)),
            pl.BlockSpec((1, S, DH), lambda h, i: (h, 0, 0)),
        ],
        out_specs=pl.BlockSpec((1, TQ, DH), lambda h, i: (h, i, 0)),
        out_shape=jax.ShapeDtypeStruct((H, S, DH), F32),
    )(q, k, v)
    of = o.transpose(1, 0, 2).reshape(S, D)

    x1, n2, gate = pl.pallas_call(
        _post_kernel,
        grid=(S // TM,),
        in_specs=[
            pl.BlockSpec((TM, D), lambda i: (i, 0)),
            pl.BlockSpec((TM, D), lambda i: (i, 0)),
            pl.BlockSpec((D, D), lambda i: (0, 0)),
            pl.BlockSpec((1, D), lambda i: (0, 0)),
            pl.BlockSpec((1, D), lambda i: (0, 0)),
            pl.BlockSpec((1, D), lambda i: (0, 0)),
            pl.BlockSpec((NT, D), lambda i: (0, 0)),
            pl.BlockSpec((1, NT), lambda i: (0, 0)),
        ],
        out_specs=[
            pl.BlockSpec((TM, D), lambda i: (i, 0)),
            pl.BlockSpec((TM, D), lambda i: (i, 0)),
            pl.BlockSpec((TM, NT), lambda i: (i, 0)),
        ],
        out_shape=[
            jax.ShapeDtypeStruct((S, D), F32),
            jax.ShapeDtypeStruct((S, D), F32),
            jax.ShapeDtypeStruct((S, NT), F32),
        ],
    )(of, xf, out_proj_w, bo, g2, b2, gate_w, bg)

    e = jnp.argmax(gate, axis=-1).astype(jnp.int32)
    counts = jnp.sum(gate, axis=0).astype(jnp.int32)
    blocks_t = (counts + TM - 1) // TM
    ends = jnp.cumsum(blocks_t)
    blk_start = ends - blocks_t
    nblk = ends[-1].astype(jnp.int32)
    ranks = jnp.cumsum(gate, axis=0) - gate
    r = jnp.sum(ranks * gate, axis=1).astype(jnp.int32)
    slot = blk_start[e] * TM + r
    be = jnp.searchsorted(ends, jnp.arange(NB, dtype=jnp.int32), side='right')
    be = jnp.minimum(be, NT - 1).astype(jnp.int32)
    be_last = jnp.take(be, nblk - 1)
    be = jnp.where(jnp.arange(NB) < nblk, be, be_last)
    nblk_arr = nblk.reshape(1)

    xg, x1g = _dispatch_sc(n2, x1, slot)

    grid_spec = pltpu.PrefetchScalarGridSpec(
        num_scalar_prefetch=2,
        grid=(NB,),
        in_specs=[
            pl.BlockSpec((TM, D), lambda b, be, nb: (b, 0)),
            pl.BlockSpec((TM, D), lambda b, be, nb: (b, 0)),
            pl.BlockSpec((1, D, DFF), lambda b, be, nb: (be[b], 0, 0)),
            pl.BlockSpec((1, 1, DFF), lambda b, be, nb: (be[b], 0, 0)),
            pl.BlockSpec((1, DFF, D), lambda b, be, nb: (be[b], 0, 0)),
            pl.BlockSpec((1, 1, D), lambda b, be, nb: (be[b], 0, 0)),
        ],
        out_specs=pl.BlockSpec((TM, D), lambda b, be, nb: (b, 0)),
    )
    yg = pl.pallas_call(
        _ffn_kernel,
        grid_spec=grid_spec,
        out_shape=jax.ShapeDtypeStruct((NB * TM, D), F32),
        compiler_params=pltpu.CompilerParams(
            dimension_semantics=("arbitrary",)),
    )(be, nblk_arr, xg, x1g, w_up, bu, w_down, bd)

    out = _combine_sc(yg, slot)

    return out.reshape(1, S, D), gate.reshape(1, S, NT)


# speedup vs baseline: 2.3515x; 1.1487x over previous
import functools

import jax
import jax.numpy as jnp
import numpy as np
from jax import lax
from jax.experimental import pallas as pl
from jax.experimental.pallas import tpu as pltpu
from jax.experimental.pallas import tpu_sc as plsc

S, D, H, NT = 2048, 768, 12, 8
DFF = 4 * D
DH = D // H
TM = 256
TQ = 512
NB = S // TM + NT - 1
EPS = 1e-5
F32 = jnp.float32

_SC_INFO = plsc.get_sparse_core_info()
_NW = _SC_INFO.num_cores * _SC_INFO.num_subcores
_TPW = S // _NW


def _ln(x, g, b):
    m = jnp.mean(x, axis=-1, keepdims=True)
    v = jnp.mean((x - m) ** 2, axis=-1, keepdims=True)
    return (x - m) / jnp.sqrt(v + EPS) * g + b


def _qkv_kernel(x_ref, g_ref, b_ref, w_ref, wb_ref, q_ref, k_ref, v_ref):
    n = _ln(x_ref[...], g_ref[...], b_ref[...])
    qkv = jax.lax.dot_general(
        n, w_ref[...], (((1,), (1,)), ((), ())),
        preferred_element_type=F32) + wb_ref[...]
    qkv = qkv.reshape(TM, 3, H, DH)
    q_ref[...] = jnp.transpose(qkv[:, 0], (1, 0, 2))
    k_ref[...] = jnp.transpose(qkv[:, 1], (1, 0, 2))
    v_ref[...] = jnp.transpose(qkv[:, 2], (1, 0, 2))


def _attn_kernel(q_ref, k_ref, v_ref, o_ref):
    s = jax.lax.dot_general(q_ref[0], k_ref[0], (((1,), (1,)), ((), ())),
                            preferred_element_type=F32) * (1.0 / np.sqrt(DH))
    m = jnp.max(s, axis=-1, keepdims=True)
    p = jnp.exp(s - m)
    p = p / jnp.sum(p, axis=-1, keepdims=True)
    o_ref[0] = jnp.dot(p, v_ref[0], preferred_element_type=F32)


def _post_kernel(o_ref, x_ref, wo_ref, bo_ref, g2_ref, b2_ref, wg_ref, bg_ref,
                 x1_ref, n2_ref, gate_ref):
    of = jnp.transpose(o_ref[...], (1, 0, 2)).reshape(TM, D)
    a = jax.lax.dot_general(of, wo_ref[...], (((1,), (1,)), ((), ())),
                            preferred_element_type=F32) + bo_ref[...]
    x1 = x_ref[...] + a
    x1_ref[...] = x1
    n2 = _ln(x1, g2_ref[...], b2_ref[...])
    n2_ref[...] = n2
    logits = jax.lax.dot_general(n2, wg_ref[...], (((1,), (1,)), ((), ())),
                                 preferred_element_type=F32) + bg_ref[...]
    mx = jnp.max(logits, axis=-1, keepdims=True)
    iot = jax.lax.broadcasted_iota(jnp.int32, logits.shape, 1)
    first = jnp.min(jnp.where(logits >= mx, iot, NT), axis=-1, keepdims=True)
    gate_ref[...] = (iot == first).astype(F32)


def _dispatch_sc(n2, x1, slot):
    mesh = plsc.VectorSubcoreMesh(core_axis_name="c", subcore_axis_name="s")

    @functools.partial(
        pl.kernel, mesh=mesh,
        out_type=[jax.ShapeDtypeStruct((NB * TM, D), F32),
                  jax.ShapeDtypeStruct((NB * TM, D), F32)],
        scratch_types=[pltpu.VMEM((_TPW,), jnp.int32),
                       pltpu.VMEM((_TPW, D), F32),
                       pltpu.SemaphoreType.DMA],
    )
    def k(n2_hbm, x1_hbm, slot_hbm, xg_hbm, x1g_hbm, idx_v, rows_v, sem):
        wid = lax.axis_index("s") * _SC_INFO.num_cores + lax.axis_index("c")
        base = wid * _TPW
        pltpu.sync_copy(slot_hbm.at[pl.ds(base, _TPW)], idx_v)
        pltpu.sync_copy(n2_hbm.at[pl.ds(base, _TPW)], rows_v)
        pltpu.async_copy(rows_v, xg_hbm.at[idx_v], sem).wait()
        pltpu.sync_copy(x1_hbm.at[pl.ds(base, _TPW)], rows_v)
        pltpu.async_copy(rows_v, x1g_hbm.at[idx_v], sem).wait()

    return k(n2, x1, slot)


def _combine_sc(yg, slot):
    mesh = plsc.VectorSubcoreMesh(core_axis_name="c", subcore_axis_name="s")

    @functools.partial(
        pl.kernel, mesh=mesh,
        out_type=jax.ShapeDtypeStruct((S, D), F32),
        scratch_types=[pltpu.VMEM((_TPW,), jnp.int32),
                       pltpu.VMEM((_TPW, D), F32),
                       pltpu.SemaphoreType.DMA],
    )
    def k(yg_hbm, slot_hbm, out_hbm, idx_v, rows_v, sem):
        wid = lax.axis_index("s") * _SC_INFO.num_cores + lax.axis_index("c")
        base = wid * _TPW
        pltpu.sync_copy(slot_hbm.at[pl.ds(base, _TPW)], idx_v)
        pltpu.async_copy(yg_hbm.at[idx_v], rows_v, sem).wait()
        pltpu.sync_copy(rows_v, out_hbm.at[pl.ds(base, _TPW)])

    return k(yg, slot)


def _ffn_kernel(be_ref, nblk_ref, xg_ref, x1g_ref,
                wu_ref, bu_ref, wd_ref, bd_ref, yg_ref):
    del be_ref
    b = pl.program_id(0)

    @pl.when(b < nblk_ref[0])
    def _():
        h = jnp.maximum(
            jnp.dot(xg_ref[...], wu_ref[0], preferred_element_type=F32)
            + bu_ref[0], 0.0)
        yg_ref[...] = (jnp.dot(h, wd_ref[0], preferred_element_type=F32)
                       + bd_ref[0] + x1g_ref[...])


def kernel(x, ln1_g, ln1_b, in_proj_w, in_proj_b, out_proj_w, out_proj_b,
           ln2_g, ln2_b, gate_w, gate_b, w_up, b_up, w_down, b_down):
    xf = x.reshape(S, D)
    bu = b_up.reshape(NT, 1, DFF)
    bd = b_down.reshape(NT, 1, D)
    g1 = ln1_g.reshape(1, D); b1 = ln1_b.reshape(1, D)
    g2 = ln2_g.reshape(1, D); b2 = ln2_b.reshape(1, D)
    bqkv = in_proj_b.reshape(1, 3 * D)
    bo = out_proj_b.reshape(1, D)
    bg = gate_b.reshape(1, NT)

    qkv = pl.pallas_call(
        _qkv_kernel,
        grid=(S // TM,),
        in_specs=[
            pl.BlockSpec((TM, D), lambda i: (i, 0)),
            pl.BlockSpec((1, D), lambda i: (0, 0)),
            pl.BlockSpec((1, D), lambda i: (0, 0)),
            pl.BlockSpec((3 * D, D), lambda i: (0, 0)),
            pl.BlockSpec((1, 3 * D), lambda i: (0, 0)),
        ],
        out_specs=[
            pl.BlockSpec((H, TM, DH), lambda i: (0, i, 0)),
            pl.BlockSpec((H, TM, DH), lambda i: (0, i, 0)),
            pl.BlockSpec((H, TM, DH), lambda i: (0, i, 0)),
        ],
        out_shape=[
            jax.ShapeDtypeStruct((H, S, DH), F32),
            jax.ShapeDtypeStruct((H, S, DH), F32),
            jax.ShapeDtypeStruct((H, S, DH), F32),
        ],
    )(xf, g1, b1, in_proj_w, bqkv)
    q, k, v = qkv

    o = pl.pallas_call(
        _attn_kernel,
        grid=(H, S // TQ),
        in_specs=[
            pl.BlockSpec((1, TQ, DH), lambda h, i: (h, i, 0)),
            pl.BlockSpec((1, S, DH), lambda h, i: (h, 0, 0)),
            pl.BlockSpec((1, S, DH), lambda h, i: (h, 0, 0)),
        ],
        out_specs=pl.BlockSpec((1, TQ, DH), lambda h, i: (h, i, 0)),
        out_shape=jax.ShapeDtypeStruct((H, S, DH), F32),
    )(q, k, v)

    x1, n2, gate = pl.pallas_call(
        _post_kernel,
        grid=(S // TM,),
        in_specs=[
            pl.BlockSpec((H, TM, DH), lambda i: (0, i, 0)),
            pl.BlockSpec((TM, D), lambda i: (i, 0)),
            pl.BlockSpec((D, D), lambda i: (0, 0)),
            pl.BlockSpec((1, D), lambda i: (0, 0)),
            pl.BlockSpec((1, D), lambda i: (0, 0)),
            pl.BlockSpec((1, D), lambda i: (0, 0)),
            pl.BlockSpec((NT, D), lambda i: (0, 0)),
            pl.BlockSpec((1, NT), lambda i: (0, 0)),
        ],
        out_specs=[
            pl.BlockSpec((TM, D), lambda i: (i, 0)),
            pl.BlockSpec((TM, D), lambda i: (i, 0)),
            pl.BlockSpec((TM, NT), lambda i: (i, 0)),
        ],
        out_shape=[
            jax.ShapeDtypeStruct((S, D), F32),
            jax.ShapeDtypeStruct((S, D), F32),
            jax.ShapeDtypeStruct((S, NT), F32),
        ],
    )(o, xf, out_proj_w, bo, g2, b2, gate_w, bg)

    e = jnp.argmax(gate, axis=-1).astype(jnp.int32)
    counts = jnp.sum(gate, axis=0).astype(jnp.int32)
    blocks_t = (counts + TM - 1) // TM
    ends = jnp.cumsum(blocks_t)
    blk_start = ends - blocks_t
    nblk = ends[-1].astype(jnp.int32)
    ranks = jnp.cumsum(gate, axis=0) - gate
    r = jnp.sum(ranks * gate, axis=1).astype(jnp.int32)
    slot = blk_start[e] * TM + r
    be = jnp.searchsorted(ends, jnp.arange(NB, dtype=jnp.int32), side='right')
    be = jnp.minimum(be, NT - 1).astype(jnp.int32)
    be_last = jnp.take(be, nblk - 1)
    be = jnp.where(jnp.arange(NB) < nblk, be, be_last)
    nblk_arr = nblk.reshape(1)

    xg, x1g = _dispatch_sc(n2, x1, slot)

    grid_spec = pltpu.PrefetchScalarGridSpec(
        num_scalar_prefetch=2,
        grid=(NB,),
        in_specs=[
            pl.BlockSpec((TM, D), lambda b, be, nb: (b, 0)),
            pl.BlockSpec((TM, D), lambda b, be, nb: (b, 0)),
            pl.BlockSpec((1, D, DFF), lambda b, be, nb: (be[b], 0, 0)),
            pl.BlockSpec((1, 1, DFF), lambda b, be, nb: (be[b], 0, 0)),
            pl.BlockSpec((1, DFF, D), lambda b, be, nb: (be[b], 0, 0)),
            pl.BlockSpec((1, 1, D), lambda b, be, nb: (be[b], 0, 0)),
        ],
        out_specs=pl.BlockSpec((TM, D), lambda b, be, nb: (b, 0)),
    )
    yg = pl.pallas_call(
        _ffn_kernel,
        grid_spec=grid_spec,
        out_shape=jax.ShapeDtypeStruct((NB * TM, D), F32),
        compiler_params=pltpu.CompilerParams(
            dimension_semantics=("arbitrary",)),
    )(be, nblk_arr, xg, x1g, w_up, bu, w_down, bd)

    out = _combine_sc(yg, slot)

    return out.reshape(1, S, D), gate.reshape(1, S, NT)


# speedup vs baseline: 2.4864x; 1.0574x over previous
import functools

import jax
import jax.numpy as jnp
import numpy as np
from jax import lax
from jax.experimental import pallas as pl
from jax.experimental.pallas import tpu as pltpu
from jax.experimental.pallas import tpu_sc as plsc

S, D, H, NT = 2048, 768, 12, 8
DFF = 4 * D
DH = D // H
TM = 256
TQ = 1024
NB = S // TM + NT - 1
EPS = 1e-5
F32 = jnp.float32

_SC_INFO = plsc.get_sparse_core_info()
_NW = _SC_INFO.num_cores * _SC_INFO.num_subcores
_TPW = S // _NW


def _ln(x, g, b):
    m = jnp.mean(x, axis=-1, keepdims=True)
    v = jnp.mean((x - m) ** 2, axis=-1, keepdims=True)
    return (x - m) / jnp.sqrt(v + EPS) * g + b


def _qkv_kernel(x_ref, g_ref, b_ref, w_ref, wb_ref, q_ref, k_ref, v_ref):
    n = _ln(x_ref[...], g_ref[...], b_ref[...])
    qkv = jax.lax.dot_general(
        n, w_ref[...], (((1,), (1,)), ((), ())),
        preferred_element_type=F32) + wb_ref[...]
    qkv = qkv.reshape(TM, 3, H, DH)
    q_ref[...] = jnp.transpose(qkv[:, 0], (1, 0, 2))
    k_ref[...] = jnp.transpose(qkv[:, 1], (1, 0, 2))
    v_ref[...] = jnp.transpose(qkv[:, 2], (1, 0, 2))


def _attn_kernel(q_ref, k_ref, v_ref, o_ref):
    s = jax.lax.dot_general(q_ref[0] * (1.0 / np.sqrt(DH)), k_ref[0],
                            (((1,), (1,)), ((), ())),
                            preferred_element_type=F32)
    m = jnp.max(s, axis=-1, keepdims=True)
    p = jnp.exp(s - m)
    p = p / jnp.sum(p, axis=-1, keepdims=True)
    o_ref[0] = jnp.dot(p, v_ref[0], preferred_element_type=F32)


def _post_kernel(o_ref, x_ref, wo_ref, bo_ref, g2_ref, b2_ref, wg_ref, bg_ref,
                 x1_ref, n2_ref, gate_ref):
    of = jnp.transpose(o_ref[...], (1, 0, 2)).reshape(TM, D)
    a = jax.lax.dot_general(of, wo_ref[...], (((1,), (1,)), ((), ())),
                            preferred_element_type=F32) + bo_ref[...]
    x1 = x_ref[...] + a
    x1_ref[...] = x1
    n2 = _ln(x1, g2_ref[...], b2_ref[...])
    n2_ref[...] = n2
    logits = jax.lax.dot_general(n2, wg_ref[...], (((1,), (1,)), ((), ())),
                                 preferred_element_type=F32) + bg_ref[...]
    mx = jnp.max(logits, axis=-1, keepdims=True)
    iot = jax.lax.broadcasted_iota(jnp.int32, logits.shape, 1)
    first = jnp.min(jnp.where(logits >= mx, iot, NT), axis=-1, keepdims=True)
    gate_ref[...] = (iot == first).astype(F32)


def _dispatch_sc(n2, x1, slot):
    mesh = plsc.VectorSubcoreMesh(core_axis_name="c", subcore_axis_name="s")

    @functools.partial(
        pl.kernel, mesh=mesh,
        out_type=[jax.ShapeDtypeStruct((NB * TM, D), F32),
                  jax.ShapeDtypeStruct((NB * TM, D), F32)],
        scratch_types=[pltpu.VMEM((_TPW,), jnp.int32),
                       pltpu.VMEM((_TPW, D), F32),
                       pltpu.SemaphoreType.DMA],
    )
    def k(n2_hbm, x1_hbm, slot_hbm, xg_hbm, x1g_hbm, idx_v, rows_v, sem):
        wid = lax.axis_index("s") * _SC_INFO.num_cores + lax.axis_index("c")
        base = wid * _TPW
        pltpu.sync_copy(slot_hbm.at[pl.ds(base, _TPW)], idx_v)
        pltpu.sync_copy(n2_hbm.at[pl.ds(base, _TPW)], rows_v)
        pltpu.async_copy(rows_v, xg_hbm.at[idx_v], sem).wait()
        pltpu.sync_copy(x1_hbm.at[pl.ds(base, _TPW)], rows_v)
        pltpu.async_copy(rows_v, x1g_hbm.at[idx_v], sem).wait()

    return k(n2, x1, slot)


def _combine_sc(yg, slot):
    mesh = plsc.VectorSubcoreMesh(core_axis_name="c", subcore_axis_name="s")

    @functools.partial(
        pl.kernel, mesh=mesh,
        out_type=jax.ShapeDtypeStruct((S, D), F32),
        scratch_types=[pltpu.VMEM((_TPW,), jnp.int32),
                       pltpu.VMEM((_TPW, D), F32),
                       pltpu.SemaphoreType.DMA],
    )
    def k(yg_hbm, slot_hbm, out_hbm, idx_v, rows_v, sem):
        wid = lax.axis_index("s") * _SC_INFO.num_cores + lax.axis_index("c")
        base = wid * _TPW
        pltpu.sync_copy(slot_hbm.at[pl.ds(base, _TPW)], idx_v)
        pltpu.async_copy(yg_hbm.at[idx_v], rows_v, sem).wait()
        pltpu.sync_copy(rows_v, out_hbm.at[pl.ds(base, _TPW)])

    return k(yg, slot)


def _ffn_kernel(be_ref, nblk_ref, xg_ref, x1g_ref,
                wu_ref, bu_ref, wd_ref, bd_ref, yg_ref):
    del be_ref
    b = pl.program_id(0)

    @pl.when(b < nblk_ref[0])
    def _():
        h = jnp.maximum(
            jnp.dot(xg_ref[...], wu_ref[0], preferred_element_type=F32)
            + bu_ref[0], 0.0)
        yg_ref[...] = (jnp.dot(h, wd_ref[0], preferred_element_type=F32)
                       + bd_ref[0] + x1g_ref[...])


def kernel(x, ln1_g, ln1_b, in_proj_w, in_proj_b, out_proj_w, out_proj_b,
           ln2_g, ln2_b, gate_w, gate_b, w_up, b_up, w_down, b_down):
    xf = x.reshape(S, D)
    bu = b_up.reshape(NT, 1, DFF)
    bd = b_down.reshape(NT, 1, D)
    g1 = ln1_g.reshape(1, D); b1 = ln1_b.reshape(1, D)
    g2 = ln2_g.reshape(1, D); b2 = ln2_b.reshape(1, D)
    bqkv = in_proj_b.reshape(1, 3 * D)
    bo = out_proj_b.reshape(1, D)
    bg = gate_b.reshape(1, NT)

    qkv = pl.pallas_call(
        _qkv_kernel,
        grid=(S // TM,),
        in_specs=[
            pl.BlockSpec((TM, D), lambda i: (i, 0)),
            pl.BlockSpec((1, D), lambda i: (0, 0)),
            pl.BlockSpec((1, D), lambda i: (0, 0)),
            pl.BlockSpec((3 * D, D), lambda i: (0, 0)),
            pl.BlockSpec((1, 3 * D), lambda i: (0, 0)),
        ],
        out_specs=[
            pl.BlockSpec((H, TM, DH), lambda i: (0, i, 0)),
            pl.BlockSpec((H, TM, DH), lambda i: (0, i, 0)),
            pl.BlockSpec((H, TM, DH), lambda i: (0, i, 0)),
        ],
        out_shape=[
            jax.ShapeDtypeStruct((H, S, DH), F32),
            jax.ShapeDtypeStruct((H, S, DH), F32),
            jax.ShapeDtypeStruct((H, S, DH), F32),
        ],
    )(xf, g1, b1, in_proj_w, bqkv)
    q, k, v = qkv

    o = pl.pallas_call(
        _attn_kernel,
        grid=(H, S // TQ),
        in_specs=[
            pl.BlockSpec((1, TQ, DH), lambda h, i: (h, i, 0)),
            pl.BlockSpec((1, S, DH), lambda h, i: (h, 0, 0)),
            pl.BlockSpec((1, S, DH), lambda h, i: (h, 0, 0)),
        ],
        out_specs=pl.BlockSpec((1, TQ, DH), lambda h, i: (h, i, 0)),
        out_shape=jax.ShapeDtypeStruct((H, S, DH), F32),
    )(q, k, v)

    x1, n2, gate = pl.pallas_call(
        _post_kernel,
        grid=(S // TM,),
        in_specs=[
            pl.BlockSpec((H, TM, DH), lambda i: (0, i, 0)),
            pl.BlockSpec((TM, D), lambda i: (i, 0)),
            pl.BlockSpec((D, D), lambda i: (0, 0)),
            pl.BlockSpec((1, D), lambda i: (0, 0)),
            pl.BlockSpec((1, D), lambda i: (0, 0)),
            pl.BlockSpec((1, D), lambda i: (0, 0)),
            pl.BlockSpec((NT, D), lambda i: (0, 0)),
            pl.BlockSpec((1, NT), lambda i: (0, 0)),
        ],
        out_specs=[
            pl.BlockSpec((TM, D), lambda i: (i, 0)),
            pl.BlockSpec((TM, D), lambda i: (i, 0)),
            pl.BlockSpec((TM, NT), lambda i: (i, 0)),
        ],
        out_shape=[
            jax.ShapeDtypeStruct((S, D), F32),
            jax.ShapeDtypeStruct((S, D), F32),
            jax.ShapeDtypeStruct((S, NT), F32),
        ],
    )(o, xf, out_proj_w, bo, g2, b2, gate_w, bg)

    e = jnp.argmax(gate, axis=-1).astype(jnp.int32)
    counts = jnp.sum(gate, axis=0).astype(jnp.int32)
    blocks_t = (counts + TM - 1) // TM
    ends = jnp.cumsum(blocks_t)
    blk_start = ends - blocks_t
    nblk = ends[-1].astype(jnp.int32)
    ranks = jnp.cumsum(gate, axis=0) - gate
    r = jnp.sum(ranks * gate, axis=1).astype(jnp.int32)
    slot = blk_start[e] * TM + r
    be = jnp.searchsorted(ends, jnp.arange(NB, dtype=jnp.int32), side='right')
    be = jnp.minimum(be, NT - 1).astype(jnp.int32)
    be_last = jnp.take(be, nblk - 1)
    be = jnp.where(jnp.arange(NB) < nblk, be, be_last)
    nblk_arr = nblk.reshape(1)

    xg, x1g = _dispatch_sc(n2, x1, slot)

    grid_spec = pltpu.PrefetchScalarGridSpec(
        num_scalar_prefetch=2,
        grid=(NB,),
        in_specs=[
            pl.BlockSpec((TM, D), lambda b, be, nb: (b, 0)),
            pl.BlockSpec((TM, D), lambda b, be, nb: (b, 0)),
            pl.BlockSpec((1, D, DFF), lambda b, be, nb: (be[b], 0, 0)),
            pl.BlockSpec((1, 1, DFF), lambda b, be, nb: (be[b], 0, 0)),
            pl.BlockSpec((1, DFF, D), lambda b, be, nb: (be[b], 0, 0)),
            pl.BlockSpec((1, 1, D), lambda b, be, nb: (be[b], 0, 0)),
        ],
        out_specs=pl.BlockSpec((TM, D), lambda b, be, nb: (b, 0)),
    )
    yg = pl.pallas_call(
        _ffn_kernel,
        grid_spec=grid_spec,
        out_shape=jax.ShapeDtypeStruct((NB * TM, D), F32),
        compiler_params=pltpu.CompilerParams(
            dimension_semantics=("arbitrary",)),
    )(be, nblk_arr, xg, x1g, w_up, bu, w_down, bd)

    out = _combine_sc(yg, slot)

    return out.reshape(1, S, D), gate.reshape(1, S, NT)
